# baseline (speedup 1.0000x reference)
import functools

import jax
import jax.numpy as jnp
from jax import lax
from jax.experimental import pallas as pl
from jax.experimental.pallas import tpu as pltpu
from jax.experimental.pallas import tpu_sc as plsc

N = 50000
D = 128
H = 128
O = 64
E = 300000

NC = 2
NS = 16
LANE = 16
CH = 128

EPT = 18816
EPAD = EPT * NS
EB = 3136
NBLK = EPT // EB
NCHMAX = (EB + 2 * CH) // CH + 1
CAP = NCHMAX * CH

NRANGE = 6
RPC = NRANGE // NC
RW = 8448
ACC_R = RW + 16
GLOC = RW
ZR_T = RW // NS
NOUT_R = NRANGE * RW
GARBAGE = N


def _sc_agg_body(with_cnt, *refs):
    (t0, t1, s0, d0, s1, d1) = refs[:6]
    n_out = 4 if with_cnt else 2
    outs = refs[6:6 + n_out]
    if with_cnt:
        (sblk0, sblk1, dblk0, dblk1, csrc, cdst2, gbuf0, gbuf1,
         stage0, stage1, si0, si1, sg0, sg1, ss, acc, hist) = refs[6 + n_out:]
    else:
        (sblk0, sblk1, dblk0, dblk1, csrc, cdst2, gbuf0, gbuf1,
         stage0, stage1, si0, si1, sg0, sg1, ss, acc) = refs[6 + n_out:]
        hist = None
    tbls = (t0, t1)
    srcs = (s0, s1)
    dsts = (d0, d1)
    sblks = (sblk0, sblk1)
    dblks = (dblk0, dblk1)
    gbufs = (gbuf0, gbuf1)
    stages = (stage0, stage1)
    sis = (si0, si1)
    sgs = (sg0, sg1)

    c = lax.axis_index("c")
    s = lax.axis_index("s")

    zero16i = jnp.zeros((LANE,), jnp.int32)
    gloc16 = jnp.full((LANE,), GLOC, jnp.int32)
    iota16 = lax.iota(jnp.int32, LANE)

    def fill_gbuf0(val):
        v16 = jnp.full((LANE,), val, jnp.float32)

        def fill(r, carry):
            for j in range(D // LANE):
                gbuf0[r, pl.ds(j * LANE, LANE)] = v16
            return carry

        lax.fori_loop(0, CH, fill, 0)

    def zero_acc():
        for kz in range(ZR_T // CH):
            pltpu.sync_copy(gbuf0, acc.at[pl.ds(s * ZR_T + kz * CH, CH)])
        rem = ZR_T - (ZR_T // CH) * CH
        if rem:
            pltpu.sync_copy(gbuf0.at[pl.ds(0, rem)],
                            acc.at[pl.ds(s * ZR_T + (ZR_T // CH) * CH, rem)])
        plsc.subcore_barrier()

    def copy_out(out_ref, rng):
        plsc.subcore_barrier()
        pltpu.sync_copy(acc.at[pl.ds(s * ZR_T, ZR_T)],
                        out_ref.at[pl.ds(rng * RW + s * ZR_T, ZR_T)])
        plsc.subcore_barrier()

    def fire_idx(rel, blk, par):
        ebase = s * EPT + blk * EB
        pltpu.async_copy(srcs[rel].at[pl.ds(ebase, EB)], sblks[par], sis[par])
        pltpu.async_copy(dsts[rel].at[pl.ds(ebase, EB)], dblks[par], sis[par])

    def wait_idx(rel, par):
        pltpu.make_async_copy(srcs[rel].at[pl.ds(0, EB)], sblks[par],
                              sis[par]).wait()
        pltpu.make_async_copy(dsts[rel].at[pl.ds(0, EB)], dblks[par],
                              sis[par]).wait()

    one16f = jnp.ones((LANE,), jnp.float32)

    def compact_block(lo, par):
        sb = sblks[par]
        db = dblks[par]

        def step(i, cur):
            sl = pl.ds(i * LANE, LANE)
            d16 = db[sl]
            s16 = sb[sl]
            m = (d16 >= lo) & (d16 < lo + RW)
            mi = m.astype(jnp.int32)
            cs = plsc.cumsum(mi)
            offs = cur + cs - mi
            plsc.store_scatter(csrc, [offs], s16, mask=m)
            plsc.store_scatter(cdst2, [offs >> 7, offs & 127], d16 - lo,
                               mask=m)
            if hist is not None:
                plsc.addupdate_scatter(hist, [d16 - lo], one16f, mask=m)
            return cur + jnp.sum(mi)

        k = lax.fori_loop(0, EB // LANE, step, 0)
        for j in range(2 * CH // LANE):
            idx = k + j * LANE + iota16
            plsc.store_scatter(csrc, [idx], zero16i)
            plsc.store_scatter(cdst2, [idx >> 7, idx & 127], gloc16)
        return jnp.maximum((k + 2 * CH - 1) // (2 * CH), 1) * 2

    def stage_to(g, par):
        for j in range(CH // LANE):
            sl = pl.ds(j * LANE, LANE)
            stages[par][sl] = cdst2[g, sl]

    def run_agg_blocks(rel, lo):
        fire_idx(rel, 0, 0)

        def bpair(b, carry):
          for q in range(2):
            blk = 2 * b + q
            par = q
            wait_idx(rel, par)

            @pl.when(blk + 1 < NBLK)
            def _():
                fire_idx(rel, blk + 1, 1 - par)

            nch = compact_block(lo, par)
            pltpu.async_copy(tbls[rel].at[csrc.at[pl.ds(0, CH)]],
                             gbufs[0], sgs[0]).wait()

            def pair(h, carry):
                g0 = 2 * h
                g1 = g0 + 1
                d1 = pltpu.async_copy(
                    tbls[rel].at[csrc.at[pl.ds(g1 * CH, CH)]], gbufs[1],
                    sgs[1])
                stage_to(g0, 0)
                pltpu.sync_copy(gbufs[0], acc.at[stages[0]], add=True)
                d1.wait()
                d2 = pltpu.async_copy(
                    tbls[rel].at[csrc.at[pl.ds((g0 + 2) * CH, CH)]],
                    gbufs[0], sgs[0])
                stage_to(g1, 1)
                pltpu.sync_copy(gbufs[1], acc.at[stages[1]], add=True)
                d2.wait()
                return carry

            lax.fori_loop(0, nch // 2 - 1, pair, 0)
            d1 = pltpu.async_copy(
                tbls[rel].at[csrc.at[pl.ds((nch - 1) * CH, CH)]], gbufs[1],
                sgs[1])
            stage_to(nch - 2, 0)
            pltpu.sync_copy(gbufs[0], acc.at[stages[0]], add=True)
            d1.wait()
            stage_to(nch - 1, 1)
            pltpu.sync_copy(gbufs[1], acc.at[stages[1]], add=True)
          return carry

        lax.fori_loop(0, NBLK // 2, bpair, 0)

    zero16f = jnp.zeros((LANE,), jnp.float32)

    def zero_hist(i, carry):
        hist[pl.ds(i * LANE, LANE)] = zero16f
        return carry

    def run_pass(rel, out_ref, hist_out):
        def pbody(p, carry):
            rng = RPC * c + p
            lo = rng * RW
            fill_gbuf0(0.0)
            zero_acc()
            if hist is not None:
                lax.fori_loop(0, RW // LANE, zero_hist, 0)
            run_agg_blocks(rel, lo)
            if hist is not None:
                pltpu.sync_copy(hist, hist_out.at[s, pl.ds(rng * RW, RW)])
            copy_out(out_ref, rng)
            return carry

        lax.fori_loop(0, RPC, pbody, 0)

    for rel in range(2):
        run_pass(rel, outs[rel], outs[2 + rel] if with_cnt else None)


def _make_sc_agg(with_cnt):
    out_type = [jax.ShapeDtypeStruct((NOUT_R, D), jnp.float32)] * 2
    if with_cnt:
        out_type += [jax.ShapeDtypeStruct((NS, NOUT_R), jnp.float32)] * 2
    mesh = plsc.VectorSubcoreMesh(core_axis_name="c", subcore_axis_name="s")
    return pl.kernel(
        functools.partial(_sc_agg_body, with_cnt),
        out_type=out_type,
        mesh=mesh,
        scratch_types=[
            pltpu.VMEM((EB,), jnp.int32),
            pltpu.VMEM((EB,), jnp.int32),
            pltpu.VMEM((EB,), jnp.int32),
            pltpu.VMEM((EB,), jnp.int32),
            pltpu.VMEM((CAP,), jnp.int32),
            pltpu.VMEM((NCHMAX, CH), jnp.int32),
            pltpu.VMEM((CH, D), jnp.float32),
            pltpu.VMEM((CH, D), jnp.float32),
            pltpu.VMEM((CH,), jnp.int32),
            pltpu.VMEM((CH,), jnp.int32),
            pltpu.SemaphoreType.DMA,
            pltpu.SemaphoreType.DMA,
            pltpu.SemaphoreType.DMA,
            pltpu.SemaphoreType.DMA,
            pltpu.SemaphoreType.DMA,
            pltpu.VMEM_SHARED((ACC_R, D), jnp.float32),
        ] + ([pltpu.VMEM((RW,), jnp.float32)] if with_cnt else []),
        compiler_params=pltpu.CompilerParams(needs_layout_passes=False),
        name="sc_agg_cnt" if with_cnt else "sc_agg",
    )


_sc_agg_l0 = _make_sc_agg(with_cnt=True)
_sc_agg_l1 = _make_sc_agg(with_cnt=False)


def _make_dense(dout, relu):
    BR = 512
    grid = ((N + BR - 1) // BR,)

    def body(agg_ref, cnt_ref, xd_ref, wl_ref, bl_ref, wr_ref, o_ref):
        cnt = jnp.maximum(jnp.sum(cnt_ref[...], axis=0), 1.0)
        mean = agg_ref[...] / cnt[:, None]
        acc = jnp.dot(xd_ref[...], wr_ref[...],
                      preferred_element_type=jnp.float32)
        acc = acc + jnp.dot(mean, wl_ref[...],
                            preferred_element_type=jnp.float32)
        r = acc + bl_ref[...]
        if relu:
            r = jnp.maximum(r, 0.0)
        o_ref[...] = r

    blk = lambda i: (i, 0)
    fix = lambda i: (0, 0)
    return pl.pallas_call(
        body,
        grid=grid,
        in_specs=[
            pl.BlockSpec((BR, D), blk),
            pl.BlockSpec((NS, BR), lambda i: (0, i)),
            pl.BlockSpec((BR, D), blk),
            pl.BlockSpec((D, dout), fix),
            pl.BlockSpec((1, dout), fix),
            pl.BlockSpec((D, dout), fix),
        ],
        out_specs=pl.BlockSpec((BR, dout), blk),
        out_shape=jax.ShapeDtypeStruct((N, dout), jnp.float32),
    )


_dense_l0 = _make_dense(H, relu=True)
_dense_l1 = _make_dense(O, relu=False)


def _pad_edges(ei):
    src = ei[0]
    dst = ei[1]
    pad = EPAD - E
    src_p = jnp.concatenate([src, jnp.zeros((pad,), jnp.int32)])
    dst_p = jnp.concatenate([dst, jnp.full((pad,), GARBAGE, jnp.int32)])
    return src_p, dst_p


def kernel(x_user, x_item, edge_index_u2i, edge_index_i2u,
           Wl0_u2i, bl0_u2i, Wr0_u2i, Wl0_i2u, bl0_i2u, Wr0_i2u,
           Wl1_u2i, bl1_u2i, Wr1_u2i, Wl1_i2u, bl1_i2u, Wr1_i2u):
    srcu, dstu = _pad_edges(edge_index_u2i)
    srci, dsti = _pad_edges(edge_index_i2u)
    aggu, aggi, cntu, cnti = _sc_agg_l0(x_user, x_item, srcu, dstu,
                                        srci, dsti)

    item1 = _dense_l0(aggu, cntu, x_item,
                      Wl0_u2i, bl0_u2i.reshape(1, H), Wr0_u2i)
    user1 = _dense_l0(aggi, cnti, x_user,
                      Wl0_i2u, bl0_i2u.reshape(1, H), Wr0_i2u)

    aggu2, aggi2 = _sc_agg_l1(user1, item1, srcu, dstu, srci, dsti)

    item2 = _dense_l1(aggu2, cntu, item1,
                      Wl1_u2i, bl1_u2i.reshape(1, O), Wr1_u2i)
    user2 = _dense_l1(aggi2, cnti, user1,
                      Wl1_i2u, bl1_i2u.reshape(1, O), Wr1_i2u)
    return (user2, item2)


# speedup vs baseline: 1.9746x; 1.9746x over previous
import functools

import jax
import jax.numpy as jnp
from jax import lax
from jax.experimental import pallas as pl
from jax.experimental.pallas import tpu as pltpu
from jax.experimental.pallas import tpu_sc as plsc

N = 50000
D = 128
H = 128
O = 64
E = 300000

NC = 2
NS = 16
LANE = 16
CH = 128

EPT = 18816
EPAD = EPT * NS
EB = 3136
NBLK = EPT // EB
NCHMAX = (EB + 2 * CH) // CH + 1
CAP = NCHMAX * CH

NRANGE = 6
RPC = NRANGE // NC
RW = 8448
ACC_R = RW + 16
GLOC = RW
ZR_T = RW // NS
NOUT_R = NRANGE * RW
GARBAGE = N


def _sc_agg_body(with_cnt, *refs):
    (t0, t1, s0, d0, s1, d1) = refs[:6]
    n_out = 4 if with_cnt else 2
    outs = refs[6:6 + n_out]
    if with_cnt:
        (sblk0, sblk1, dblk0, dblk1, csrc, cdst2, gbuf0, gbuf1,
         stage0, stage1, si0, si1, sg0, sg1, ss, acc, hist) = refs[6 + n_out:]
    else:
        (sblk0, sblk1, dblk0, dblk1, csrc, cdst2, gbuf0, gbuf1,
         stage0, stage1, si0, si1, sg0, sg1, ss, acc) = refs[6 + n_out:]
        hist = None
    tbls = (t0, t1)
    srcs = (s0, s1)
    dsts = (d0, d1)
    sblks = (sblk0, sblk1)
    dblks = (dblk0, dblk1)
    gbufs = (gbuf0, gbuf1)
    stages = (stage0, stage1)
    sis = (si0, si1)
    sgs = (sg0, sg1)

    c = lax.axis_index("c")
    s = lax.axis_index("s")

    zero16i = jnp.zeros((LANE,), jnp.int32)
    gloc16 = jnp.full((LANE,), GLOC, jnp.int32)
    iota16 = lax.iota(jnp.int32, LANE)

    def fill_gbuf0(val):
        v16 = jnp.full((LANE,), val, jnp.float32)

        def fill(r, carry):
            for j in range(D // LANE):
                gbuf0[r, pl.ds(j * LANE, LANE)] = v16
            return carry

        lax.fori_loop(0, CH, fill, 0)

    def zero_acc():
        for kz in range(ZR_T // CH):
            pltpu.sync_copy(gbuf0, acc.at[pl.ds(s * ZR_T + kz * CH, CH)])
        rem = ZR_T - (ZR_T // CH) * CH
        if rem:
            pltpu.sync_copy(gbuf0.at[pl.ds(0, rem)],
                            acc.at[pl.ds(s * ZR_T + (ZR_T // CH) * CH, rem)])
        plsc.subcore_barrier()

    def copy_out(out_ref, rng):
        plsc.subcore_barrier()
        pltpu.sync_copy(acc.at[pl.ds(s * ZR_T, ZR_T)],
                        out_ref.at[pl.ds(rng * RW + s * ZR_T, ZR_T)])
        plsc.subcore_barrier()

    def fire_idx(rel, blk, par):
        ebase = s * EPT + blk * EB
        pltpu.async_copy(srcs[rel].at[pl.ds(ebase, EB)], sblks[par], sis[par])
        pltpu.async_copy(dsts[rel].at[pl.ds(ebase, EB)], dblks[par], sis[par])

    def wait_idx(rel, par):
        pltpu.make_async_copy(srcs[rel].at[pl.ds(0, EB)], sblks[par],
                              sis[par]).wait()
        pltpu.make_async_copy(dsts[rel].at[pl.ds(0, EB)], dblks[par],
                              sis[par]).wait()

    one16f = jnp.ones((LANE,), jnp.float32)

    def compact_block(lo, par):
        sb = sblks[par]
        db = dblks[par]

        def step(i, cur):
            sl = pl.ds(i * LANE, LANE)
            d16 = db[sl]
            s16 = sb[sl]
            m = (d16 >= lo) & (d16 < lo + RW)
            mi = m.astype(jnp.int32)
            cs = plsc.cumsum(mi)
            offs = cur + cs - mi
            plsc.store_scatter(csrc, [offs], s16, mask=m)
            plsc.store_scatter(cdst2, [offs >> 7, offs & 127], d16 - lo,
                               mask=m)
            return cur + jnp.sum(mi)

        k = lax.fori_loop(0, EB // LANE, step, 0)
        for j in range(CH // LANE):
            idx = k + j * LANE + iota16
            plsc.store_scatter(csrc, [idx], zero16i)
            plsc.store_scatter(cdst2, [idx >> 7, idx & 127], gloc16)
        return (k + CH - 1) // CH

    def stage_to(g, par):
        for j in range(CH // LANE):
            sl = pl.ds(j * LANE, LANE)
            stages[par][sl] = cdst2[g, sl]

    def run_agg_blocks(rel, lo):
        fire_idx(rel, 0, 0)

        def bpair(b, carry):
          for q in range(2):
            blk = 2 * b + q
            par = q
            wait_idx(rel, par)

            @pl.when(blk + 1 < NBLK)
            def _():
                fire_idx(rel, blk + 1, 1 - par)

            nch = compact_block(lo, par)

            if hist is not None:
                def hstep(g2, carry):
                    for j in range(CH // LANE):
                        v = cdst2[g2, pl.ds(j * LANE, LANE)]
                        plsc.addupdate_scatter(hist, [v], one16f)
                    return carry

                lax.fori_loop(0, nch, hstep, 0)

            def chunk(g, carry):
                pltpu.async_copy(tbls[rel].at[csrc.at[pl.ds(g * CH, CH)]],
                                 gbufs[0], sgs[0]).wait()
                stage_to(g, 0)
                pltpu.sync_copy(gbufs[0], acc.at[stages[0]], add=True)
                return carry

            lax.fori_loop(0, nch, chunk, 0)
          return carry

        lax.fori_loop(0, NBLK // 2, bpair, 0)

    zero16f = jnp.zeros((LANE,), jnp.float32)

    def zero_hist(i, carry):
        hist[pl.ds(i * LANE, LANE)] = zero16f
        return carry

    def run_pass(rel, out_ref, hist_out):
        def pbody(p, carry):
            rng = RPC * c + p
            lo = rng * RW
            fill_gbuf0(0.0)
            zero_acc()
            if hist is not None:
                lax.fori_loop(0, RW // LANE + 1, zero_hist, 0)
            run_agg_blocks(rel, lo)
            if hist is not None:
                pltpu.sync_copy(hist.at[pl.ds(0, RW)],
                                hist_out.at[s, pl.ds(rng * RW, RW)])
            copy_out(out_ref, rng)
            return carry

        lax.fori_loop(0, RPC, pbody, 0)

    for rel in range(2):
        run_pass(rel, outs[rel], outs[2 + rel] if with_cnt else None)


def _make_sc_agg(with_cnt):
    out_type = [jax.ShapeDtypeStruct((NOUT_R, D), jnp.float32)] * 2
    if with_cnt:
        out_type += [jax.ShapeDtypeStruct((NS, NOUT_R), jnp.float32)] * 2
    mesh = plsc.VectorSubcoreMesh(core_axis_name="c", subcore_axis_name="s")
    return pl.kernel(
        functools.partial(_sc_agg_body, with_cnt),
        out_type=out_type,
        mesh=mesh,
        scratch_types=[
            pltpu.VMEM((EB,), jnp.int32),
            pltpu.VMEM((EB,), jnp.int32),
            pltpu.VMEM((EB,), jnp.int32),
            pltpu.VMEM((EB,), jnp.int32),
            pltpu.VMEM((CAP,), jnp.int32),
            pltpu.VMEM((NCHMAX, CH), jnp.int32),
            pltpu.VMEM((CH, D), jnp.float32),
            pltpu.VMEM((CH, D), jnp.float32),
            pltpu.VMEM((CH,), jnp.int32),
            pltpu.VMEM((CH,), jnp.int32),
            pltpu.SemaphoreType.DMA,
            pltpu.SemaphoreType.DMA,
            pltpu.SemaphoreType.DMA,
            pltpu.SemaphoreType.DMA,
            pltpu.SemaphoreType.DMA,
            pltpu.VMEM_SHARED((ACC_R, D), jnp.float32),
        ] + ([pltpu.VMEM((RW + LANE,), jnp.float32)] if with_cnt else []),
        compiler_params=pltpu.CompilerParams(needs_layout_passes=False),
        name="sc_agg_cnt" if with_cnt else "sc_agg",
    )


_sc_agg_l0 = _make_sc_agg(with_cnt=True)
_sc_agg_l1 = _make_sc_agg(with_cnt=False)


def _make_dense(dout, relu):
    BR = 512
    grid = ((N + BR - 1) // BR,)

    def body(agg_ref, cnt_ref, xd_ref, wl_ref, bl_ref, wr_ref, o_ref):
        cnt = jnp.maximum(jnp.sum(cnt_ref[...], axis=0), 1.0)
        mean = agg_ref[...] / cnt[:, None]
        acc = jnp.dot(xd_ref[...], wr_ref[...],
                      preferred_element_type=jnp.float32)
        acc = acc + jnp.dot(mean, wl_ref[...],
                            preferred_element_type=jnp.float32)
        r = acc + bl_ref[...]
        if relu:
            r = jnp.maximum(r, 0.0)
        o_ref[...] = r

    blk = lambda i: (i, 0)
    fix = lambda i: (0, 0)
    return pl.pallas_call(
        body,
        grid=grid,
        in_specs=[
            pl.BlockSpec((BR, D), blk),
            pl.BlockSpec((NS, BR), lambda i: (0, i)),
            pl.BlockSpec((BR, D), blk),
            pl.BlockSpec((D, dout), fix),
            pl.BlockSpec((1, dout), fix),
            pl.BlockSpec((D, dout), fix),
        ],
        out_specs=pl.BlockSpec((BR, dout), blk),
        out_shape=jax.ShapeDtypeStruct((N, dout), jnp.float32),
    )


_dense_l0 = _make_dense(H, relu=True)
_dense_l1 = _make_dense(O, relu=False)


def _pad_edges(ei):
    src = ei[0]
    dst = ei[1]
    pad = EPAD - E
    src_p = jnp.concatenate([src, jnp.zeros((pad,), jnp.int32)])
    dst_p = jnp.concatenate([dst, jnp.full((pad,), GARBAGE, jnp.int32)])
    return src_p, dst_p


def kernel(x_user, x_item, edge_index_u2i, edge_index_i2u,
           Wl0_u2i, bl0_u2i, Wr0_u2i, Wl0_i2u, bl0_i2u, Wr0_i2u,
           Wl1_u2i, bl1_u2i, Wr1_u2i, Wl1_i2u, bl1_i2u, Wr1_i2u):
    srcu, dstu = _pad_edges(edge_index_u2i)
    srci, dsti = _pad_edges(edge_index_i2u)
    aggu, aggi, cntu, cnti = _sc_agg_l0(x_user, x_item, srcu, dstu,
                                        srci, dsti)

    item1 = _dense_l0(aggu, cntu, x_item,
                      Wl0_u2i, bl0_u2i.reshape(1, H), Wr0_u2i)
    user1 = _dense_l0(aggi, cnti, x_user,
                      Wl0_i2u, bl0_i2u.reshape(1, H), Wr0_i2u)

    aggu2, aggi2 = _sc_agg_l1(user1, item1, srcu, dstu, srci, dsti)

    item2 = _dense_l1(aggu2, cntu, item1,
                      Wl1_u2i, bl1_u2i.reshape(1, O), Wr1_u2i)
    user2 = _dense_l1(aggi2, cnti, user1,
                      Wl1_i2u, bl1_i2u.reshape(1, O), Wr1_i2u)
    return (user2, item2)
